# baseline (speedup 1.0000x reference)
import functools

import jax
import jax.numpy as jnp
from jax.experimental import pallas as pl
from jax.experimental.pallas import tpu as pltpu

_STACK = 4
_ROW_BLOCK = 256


def _vq_body(x_ref, proj_ref, cb_ref, out_ref, cbn_ref):
    @pl.when(pl.program_id(0) == 0)
    def _():
        cb = cb_ref[...]
        norm = jnp.sqrt(jnp.sum(cb * cb, axis=1, keepdims=True))
        cbn_ref[...] = cb / jnp.maximum(norm, 1e-12)

    xb = x_ref[0]
    xs = xb.reshape(_ROW_BLOCK, _STACK * xb.shape[1])
    p = jnp.dot(xs, proj_ref[...],
                preferred_element_type=jnp.float32)
    scores = jnp.dot(p, cbn_ref[...].T,
                     preferred_element_type=jnp.float32,
                     precision=jax.lax.Precision.HIGHEST)
    idx = jnp.argmax(scores, axis=1).astype(jnp.int32)
    j = pl.program_id(0) % (out_ref.shape[2] // _ROW_BLOCK)
    out_ref[0, 0, pl.ds(j * _ROW_BLOCK, _ROW_BLOCK)] = idx


@functools.partial(jax.jit, static_argnames=())
def kernel(x, proj, codebook):
    b, t, c = x.shape
    t_out = t // _STACK
    t_blk = _ROW_BLOCK * _STACK
    per_b = t // t_blk
    grid = b * per_b
    out = pl.pallas_call(
        _vq_body,
        grid=(grid,),
        in_specs=[
            pl.BlockSpec((1, t_blk, c),
                         lambda i: (i // per_b, i % per_b, 0)),
            pl.BlockSpec(proj.shape, lambda i: (0, 0)),
            pl.BlockSpec(codebook.shape, lambda i: (0, 0)),
        ],
        out_specs=pl.BlockSpec((1, 1, t_out), lambda i: (i // per_b, 0, 0)),
        out_shape=jax.ShapeDtypeStruct((b, 1, t_out), jnp.int32),
        scratch_shapes=[pltpu.VMEM(codebook.shape, jnp.float32)],
    )(x, proj, codebook)
    return out.reshape(b, t_out)
